# Initial kernel scaffold; baseline (speedup 1.0000x reference)
import jax
import jax.numpy as jnp
from jax.experimental import pallas as pl


def kernel(x, edge_index, W1, b1, W2, b2, W3, b3, W4, b4):
    raise NotImplementedError("write your pallas kernel here")


# speedup vs baseline: 10.1534x; 10.1534x over previous
import functools

import jax
import jax.numpy as jnp
from jax import lax
from jax.experimental import pallas as pl
from jax.experimental.pallas import tpu as pltpu
from jax.experimental.pallas import tpu_sc as plsc

F32 = jnp.float32

N = 10000
E = 320000
D = 128

NC = 2
NS = 16
NW = NC * NS
E_PER_TILE = E // NW
CHUNK = 80
NCHUNK = E_PER_TILE // CHUNK
N_PAD = 10240
ROWS_PER_TILE = N_PAD // NS
DEG_PAD = 10240
DEG_W = 16
DEG_ROWS_PER_TILE = DEG_PAD // NS

_MESH = plsc.VectorSubcoreMesh(core_axis_name="c", subcore_axis_name="s")


def _sc_deg_body(dstr, zdeg, ones_hbm,
                 out_deg,
                 dst_v, ones_v, deg_sh):
    c = lax.axis_index("c")
    s = lax.axis_index("s")
    g = c * NS + s
    pltpu.sync_copy(dstr.at[g], dst_v)
    pltpu.sync_copy(ones_hbm, ones_v)
    db = s * DEG_ROWS_PER_TILE
    pltpu.sync_copy(zdeg.at[pl.ds(db, DEG_ROWS_PER_TILE)],
                    deg_sh.at[pl.ds(db, DEG_ROWS_PER_TILE)])
    plsc.subcore_barrier()

    @pl.loop(0, NCHUNK)
    def _(j):
        pltpu.sync_copy(ones_v, deg_sh.at[dst_v.at[j]], add=True)

    plsc.subcore_barrier()
    pltpu.sync_copy(deg_sh.at[pl.ds(db, DEG_ROWS_PER_TILE)],
                    out_deg.at[c, pl.ds(db, DEG_ROWS_PER_TILE)])


def _sc_agg_body(h_hbm, srcr, dstr, zagg,
                 out_agg,
                 src_v, dst_v, rows_v, agg_sh):
    c = lax.axis_index("c")
    s = lax.axis_index("s")
    g = c * NS + s
    pltpu.sync_copy(srcr.at[g], src_v)
    pltpu.sync_copy(dstr.at[g], dst_v)
    rb = s * ROWS_PER_TILE
    pltpu.sync_copy(zagg.at[pl.ds(rb, ROWS_PER_TILE)],
                    agg_sh.at[pl.ds(rb, ROWS_PER_TILE)])
    plsc.subcore_barrier()

    @pl.loop(0, NCHUNK)
    def _(j):
        pltpu.sync_copy(h_hbm.at[src_v.at[j]], rows_v)
        pltpu.sync_copy(rows_v, agg_sh.at[dst_v.at[j]], add=True)

    plsc.subcore_barrier()
    pltpu.sync_copy(agg_sh.at[pl.ds(rb, ROWS_PER_TILE)],
                    out_agg.at[c, pl.ds(rb, ROWS_PER_TILE)])


_sc_deg = pl.kernel(
    _sc_deg_body,
    out_type=jax.ShapeDtypeStruct((NC, DEG_PAD, DEG_W), F32),
    mesh=_MESH,
    scratch_types=[
        pltpu.VMEM((NCHUNK, CHUNK), jnp.int32),
        pltpu.VMEM((CHUNK, DEG_W), F32),
        pltpu.VMEM_SHARED((DEG_PAD, DEG_W), F32),
    ],
)

_sc_agg = pl.kernel(
    _sc_agg_body,
    out_type=jax.ShapeDtypeStruct((NC, N_PAD, D), F32),
    mesh=_MESH,
    scratch_types=[
        pltpu.VMEM((NCHUNK, CHUNK), jnp.int32),
        pltpu.VMEM((NCHUNK, CHUNK), jnp.int32),
        pltpu.VMEM((CHUNK, D), F32),
        pltpu.VMEM_SHARED((N_PAD, D), F32),
    ],
)


R = 1000


def _tc1_body(h_ref, p0_ref, p1_ref, d0_ref, d1_ref, wa_ref, wb_ref, b_ref,
              out_ref, inv_ref):
    inv = 1.0 / jnp.maximum(d0_ref[...] + d1_ref[...], 1e-9)
    inv_ref[...] = inv
    agg = (p0_ref[...] + p1_ref[...]) * inv
    acc = jnp.dot(h_ref[...], wa_ref[...], preferred_element_type=F32)
    acc = acc + jnp.dot(agg, wb_ref[...], preferred_element_type=F32)
    acc = acc + b_ref[...]
    out_ref[...] = jnp.maximum(acc, 0.0)


def _tc_body(h_ref, p0_ref, p1_ref, inv_ref, wa_ref, wb_ref, b_ref, out_ref,
             *, relu):
    agg = (p0_ref[...] + p1_ref[...]) * inv_ref[...]
    acc = jnp.dot(h_ref[...], wa_ref[...], preferred_element_type=F32)
    acc = acc + jnp.dot(agg, wb_ref[...], preferred_element_type=F32)
    acc = acc + b_ref[...]
    out_ref[...] = jnp.maximum(acc, 0.0) if relu else acc


_bs_rows = pl.BlockSpec((R, D), lambda i: (i, 0))
_bs_col = pl.BlockSpec((R, 1), lambda i: (i, 0))
_bs_w = pl.BlockSpec((D, D), lambda i: (0, 0))
_bs_b = pl.BlockSpec((1, D), lambda i: (0, 0))

_tc_layer1 = pl.pallas_call(
    _tc1_body,
    grid=(N // R,),
    in_specs=[_bs_rows, _bs_rows, _bs_rows, _bs_col, _bs_col,
              _bs_w, _bs_w, _bs_b],
    out_specs=(_bs_rows, _bs_col),
    out_shape=(jax.ShapeDtypeStruct((N, D), F32),
               jax.ShapeDtypeStruct((N, 1), F32)),
)


def _make_tc_layer(relu):
    return pl.pallas_call(
        functools.partial(_tc_body, relu=relu),
        grid=(N // R,),
        in_specs=[_bs_rows, _bs_rows, _bs_rows, _bs_col,
                  _bs_w, _bs_w, _bs_b],
        out_specs=_bs_rows,
        out_shape=jax.ShapeDtypeStruct((N, D), F32),
    )


_tc_layer_relu = _make_tc_layer(True)
_tc_layer_linear = _make_tc_layer(False)


def kernel(x, edge_index, W1, b1, W2, b2, W3, b3, W4, b4):
    src_r = edge_index[0].reshape(NW, NCHUNK, CHUNK)
    dst_r = edge_index[1].reshape(NW, NCHUNK, CHUNK)
    zeros_agg = jnp.zeros((N_PAD, D), F32)
    zeros_deg = jnp.zeros((DEG_PAD, DEG_W), F32)
    ones_deg = jnp.ones((CHUNK, DEG_W), F32)

    def split(W, b):
        return W[:, :D].T, W[:, D:].T, b.reshape(1, D)

    wa1, wb1, bb1 = split(W1, b1)
    wa2, wb2, bb2 = split(W2, b2)
    wa3, wb3, bb3 = split(W3, b3)
    wa4, wb4, bb4 = split(W4, b4)

    deg_parts = _sc_deg(dst_r, zeros_deg, ones_deg)
    parts = _sc_agg(x, src_r, dst_r, zeros_agg)
    d0 = deg_parts[0, :N, 0:1]
    d1 = deg_parts[1, :N, 0:1]
    h, inv = _tc_layer1(x, parts[0], parts[1], d0, d1, wa1, wb1, bb1)

    parts = _sc_agg(h, src_r, dst_r, zeros_agg)
    h = _tc_layer_relu(h, parts[0], parts[1], inv, wa2, wb2, bb2)

    parts = _sc_agg(h, src_r, dst_r, zeros_agg)
    h = _tc_layer_relu(h, parts[0], parts[1], inv, wa3, wb3, bb3)

    parts = _sc_agg(h, src_r, dst_r, zeros_agg)
    h = _tc_layer_linear(h, parts[0], parts[1], inv, wa4, wb4, bb4)
    return h
